# baseline (speedup 1.0000x reference)
import functools

import jax
import jax.numpy as jnp
import numpy as np
from jax import lax
from jax.experimental import pallas as pl
from jax.experimental.pallas import tpu as pltpu
from jax.experimental.pallas import tpu_sc as plsc

_V = 1000
_D = 768
_L = 4
_E = 8
_S = 2048
_KTOK = max(1, int(_S * (1.0 / (1.0 + np.exp(-0.15)))))
_KEXP = max(1, int(0.8 * _E))
_VPAD = 1024
_CH = 128
_NCH = _D // _CH
_CHE = 256
_NCHE = _D // _CHE
_PREC = lax.Precision.HIGHEST


def _sc_gather(emb, idx):
    info = plsc.get_sparse_core_info()
    nw = info.num_cores * info.num_subcores
    b_per_w = _S // nw
    mesh = plsc.VectorSubcoreMesh(core_axis_name="c", subcore_axis_name="s")

    @functools.partial(
        pl.kernel,
        mesh=mesh,
        out_type=jax.ShapeDtypeStruct((_S, _D), jnp.float32),
        scratch_types=[
            pltpu.VMEM((b_per_w,), jnp.int32),
            pltpu.VMEM((b_per_w, _D), jnp.float32),
            pltpu.SemaphoreType.DMA,
        ],
    )
    def k(emb_hbm, idx_hbm, out_hbm, idx_v, rows_v, sem):
        wid = lax.axis_index("s") * info.num_cores + lax.axis_index("c")
        base = wid * b_per_w
        pltpu.sync_copy(idx_hbm.at[pl.ds(base, b_per_w)], idx_v)
        pltpu.async_copy(emb_hbm.at[idx_v], rows_v, sem).wait()
        pltpu.sync_copy(rows_v, out_hbm.at[pl.ds(base, b_per_w)])

    return k(emb, idx)


def _f32_key(x):
    u = lax.bitcast_convert_type(x, jnp.uint32)
    neg = (u >> jnp.uint32(31)) > jnp.uint32(0)
    return jnp.where(neg, ~u, u | jnp.uint32(0x80000000))


def _kth_key(keys, k):
    def bit_step(i, m):
        cand = m | (jnp.uint32(1) << (jnp.uint32(31) - i.astype(jnp.uint32)))
        cnt = jnp.sum(jnp.where(keys >= cand, 1, 0).astype(jnp.int32))
        return jnp.where(cnt >= k, cand, m)

    return lax.fori_loop(0, 32, bit_step, jnp.uint32(0))


def _route_body(h_ref, gate_ref, gwt_ref, gb_ref, hm_ref, sel_ref, hn_sc):
    gate = gate_ref[...]
    s_col = jnp.zeros((_S, 1), jnp.float32)
    for c in range(_NCH):
        sl = slice(c * _CH, (c + 1) * _CH)
        h = h_ref[:, sl]
        mean = jnp.sum(h, axis=0, keepdims=True) * (1.0 / _S)
        dlt = h - mean
        var = jnp.sum(dlt * dlt, axis=0, keepdims=True) / (_S - 1.0)
        buf = (1.0 - 0.95) * mean
        hn = jnp.where(jnp.abs(dlt) > jnp.abs(buf), mean, h)
        hn = hn / jnp.sqrt(var + 1e-6)
        hn_sc[:, sl] = hn
        s_col = s_col + lax.dot_general(
            hn, gate[sl, :], (((1,), (0,)), ((), ())),
            preferred_element_type=jnp.float32, precision=_PREC)
    keys_col = _f32_key(s_col)
    thr = _kth_key(keys_col, _KTOK)
    tok = jnp.where(keys_col > thr, 1.0, 0.0)
    gs = gb_ref[...]
    gwt = gwt_ref[...]
    for c in range(_NCH):
        sl = slice(c * _CH, (c + 1) * _CH)
        hmv = hn_sc[:, sl] * tok
        hm_ref[:, sl] = hmv
        gs = gs + lax.dot_general(
            hmv, gwt[sl, :], (((1,), (0,)), ((), ())),
            preferred_element_type=jnp.float32, precision=_PREC)
    lane = lax.broadcasted_iota(jnp.int32, (_S, _E), 1)

    def rank_step(ep, rank):
        onehot = jnp.where(lane == ep, 1.0, 0.0)
        col = jnp.sum(gs * onehot, axis=1, keepdims=True)
        rank = rank + jnp.where(col > gs, 1, 0)
        return rank + jnp.where((col == gs) & (ep < lane), 1, 0)

    rank = lax.fori_loop(0, _E, rank_step, jnp.zeros((_S, _E), jnp.int32))
    sel_ref[...] = jnp.where(rank < _KEXP, 1.0, 0.0)


def _route(h, gate2, gwt, gb2):
    return pl.pallas_call(
        _route_body,
        out_shape=(jax.ShapeDtypeStruct((_S, _D), jnp.float32),
                   jax.ShapeDtypeStruct((_S, _E), jnp.float32)),
        scratch_shapes=[pltpu.VMEM((_S, _D), jnp.float32)],
    )(h, gate2, gwt, gb2)


def _expert_body(hm_ref, sel_ref, ewt_ref, eb_ref, out_ref):
    sel = sel_ref[...]
    lane = lax.broadcasted_iota(jnp.int32, (_S, _E), 1)
    hmv = hm_ref[...]
    acc = jnp.zeros((_S, _CHE), jnp.float32)
    for e in range(_E):
        w = ewt_ref[e]
        ye = lax.dot_general(hmv, w, (((1,), (0,)), ((), ())),
                             preferred_element_type=jnp.float32,
                             precision=_PREC)
        ye = ye + eb_ref[e]
        sel_col = jnp.sum(jnp.where(lane == e, sel, 0.0), axis=1,
                          keepdims=True)
        acc = acc + sel_col * ye
    out_ref[...] = acc


def _experts(hm, sel, ewt, ebt):
    return pl.pallas_call(
        _expert_body,
        grid=(_NCHE,),
        in_specs=[
            pl.BlockSpec((_S, _D), lambda c: (0, 0)),
            pl.BlockSpec((_S, _E), lambda c: (0, 0)),
            pl.BlockSpec((_E, _D, _CHE), lambda c: (0, 0, c)),
            pl.BlockSpec((_E, 1, _CHE), lambda c: (0, 0, c)),
        ],
        out_specs=pl.BlockSpec((_S, _CHE), lambda c: (0, c)),
        out_shape=jax.ShapeDtypeStruct((_S, _D), jnp.float32),
    )(hm, sel, ewt, ebt)


def _head_body(h_ref, lnw_ref, lnb_ref, hwt_ref, hb_ref, out_ref, hn_sc):
    v = pl.program_id(0)

    @pl.when(v == 0)
    def _():
        h = h_ref[...]
        mu = jnp.mean(h, axis=1, keepdims=True)
        dlt = h - mu
        var = jnp.mean(dlt * dlt, axis=1, keepdims=True)
        hn = (h - mu) / jnp.sqrt(var + 1e-5)
        hn_sc[...] = hn * lnw_ref[...] + lnb_ref[...]

    logits = lax.dot_general(hn_sc[...], hwt_ref[...], (((1,), (0,)), ((), ())),
                             preferred_element_type=jnp.float32,
                             precision=_PREC)
    out_ref[...] = logits + hb_ref[...]


def _head_forward(h, ln_w2, ln_b2, head_wt, head_b_pad):
    vt = 256
    return pl.pallas_call(
        _head_body,
        grid=(_VPAD // vt,),
        in_specs=[
            pl.BlockSpec((_S, _D), lambda v: (0, 0)),
            pl.BlockSpec((1, _D), lambda v: (0, 0)),
            pl.BlockSpec((1, _D), lambda v: (0, 0)),
            pl.BlockSpec((_D, vt), lambda v: (0, v)),
            pl.BlockSpec((1, vt), lambda v: (0, v)),
        ],
        out_specs=pl.BlockSpec((_S, vt), lambda v: (0, v)),
        out_shape=jax.ShapeDtypeStruct((_S, _VPAD), jnp.float32),
        scratch_shapes=[pltpu.VMEM((_S, _D), jnp.float32)],
    )(h, ln_w2, ln_b2, head_wt, head_b_pad)


def kernel(x, emb, gate_w, expert_w, expert_b, gating_w, gating_b, ln_w,
           ln_b, head_w, head_b):
    idx = x.reshape(_S).astype(jnp.int32)
    h = _sc_gather(emb, idx)
    ewt = jnp.swapaxes(expert_w, -1, -2)
    for l in range(_L):
        hm, sel = _route(h, gate_w[l].reshape(1, _D).T,
                         gating_w[l].T, gating_b[l].reshape(1, _E))
        h = _experts(hm, sel, ewt[l], expert_b[l].reshape(_E, 1, _D))
    head_wt = jnp.pad(head_w, ((0, _VPAD - _V), (0, 0))).T
    head_b_pad = jnp.pad(head_b, (0, _VPAD - _V)).reshape(1, _VPAD)
    logits = _head_forward(h, ln_w.reshape(1, _D), ln_b.reshape(1, _D),
                           head_wt, head_b_pad)
    return logits[:, :_V].reshape(1, _S, _V)


# speedup vs baseline: 2.9597x; 2.9597x over previous
import functools

import jax
import jax.numpy as jnp
import numpy as np
from jax import lax
from jax.experimental import pallas as pl
from jax.experimental.pallas import tpu as pltpu
from jax.experimental.pallas import tpu_sc as plsc

_V = 1000
_D = 768
_L = 4
_E = 8
_S = 2048
_KTOK = max(1, int(_S * (1.0 / (1.0 + np.exp(-0.15)))))
_KEXP = max(1, int(0.8 * _E))
_VPAD = 1024
_CH = 256
_NCH = _D // _CH
_CHE = 256
_NCHE = _D // _CHE


def _dotk(a, b):
    acc = None
    for off in range(0, a.shape[1], 256):
        p = lax.dot_general(a[:, off:off + 256], b[off:off + 256, :],
                            (((1,), (0,)), ((), ())),
                            preferred_element_type=jnp.float32)
        acc = p if acc is None else acc + p
    return acc


def _sc_gather(emb, idx):
    info = plsc.get_sparse_core_info()
    nw = info.num_cores * info.num_subcores
    b_per_w = _S // nw
    mesh = plsc.VectorSubcoreMesh(core_axis_name="c", subcore_axis_name="s")

    @functools.partial(
        pl.kernel,
        mesh=mesh,
        out_type=jax.ShapeDtypeStruct((_S, _D), jnp.float32),
        scratch_types=[
            pltpu.VMEM((b_per_w,), jnp.int32),
            pltpu.VMEM((b_per_w, _D), jnp.float32),
            pltpu.SemaphoreType.DMA,
        ],
    )
    def k(emb_hbm, idx_hbm, out_hbm, idx_v, rows_v, sem):
        wid = lax.axis_index("s") * info.num_cores + lax.axis_index("c")
        base = wid * b_per_w
        pltpu.sync_copy(idx_hbm.at[pl.ds(base, b_per_w)], idx_v)
        pltpu.async_copy(emb_hbm.at[idx_v], rows_v, sem).wait()
        pltpu.sync_copy(rows_v, out_hbm.at[pl.ds(base, b_per_w)])

    return k(emb, idx)


def _f32_key(x):
    u = lax.bitcast_convert_type(x, jnp.uint32)
    neg = (u >> jnp.uint32(31)) > jnp.uint32(0)
    return jnp.where(neg, ~u, u | jnp.uint32(0x80000000))


def _kth_key(keys, k):
    def bit_step(i, m):
        cand = m | (jnp.uint32(1) << (jnp.uint32(31) - i.astype(jnp.uint32)))
        cnt = jnp.sum(jnp.where(keys >= cand, 1, 0).astype(jnp.int32))
        return jnp.where(cnt >= k, cand, m)

    return lax.fori_loop(0, 32, bit_step, jnp.uint32(0))


def _route_body(h_ref, gate_ref, gwt_ref, gb_ref, hm_ref, sel_ref, hn_sc):
    gate = gate_ref[...]
    s_col = jnp.zeros((_S, 1), jnp.float32)
    for c in range(_NCH):
        sl = slice(c * _CH, (c + 1) * _CH)
        h = h_ref[:, sl]
        mean = jnp.sum(h, axis=0, keepdims=True) * (1.0 / _S)
        dlt = h - mean
        var = jnp.sum(dlt * dlt, axis=0, keepdims=True) / (_S - 1.0)
        buf = (1.0 - 0.95) * mean
        hn = jnp.where(jnp.abs(dlt) > jnp.abs(buf), mean, h)
        hn = hn / jnp.sqrt(var + 1e-6)
        hn_sc[:, sl] = hn
        s_col = s_col + lax.dot_general(
            hn, gate[sl, :], (((1,), (0,)), ((), ())),
            preferred_element_type=jnp.float32)
    keys_col = _f32_key(s_col)
    thr = _kth_key(keys_col, _KTOK)
    tok = jnp.where(keys_col > thr, 1.0, 0.0)
    gs = gb_ref[...]
    gwt = gwt_ref[...]
    for c in range(_NCH):
        sl = slice(c * _CH, (c + 1) * _CH)
        hmv = hn_sc[:, sl] * tok
        hm_ref[:, sl] = hmv
        gs = gs + lax.dot_general(
            hmv, gwt[sl, :], (((1,), (0,)), ((), ())),
            preferred_element_type=jnp.float32)
    lane = lax.broadcasted_iota(jnp.int32, (_S, _E), 1)

    def rank_step(ep, rank):
        onehot = jnp.where(lane == ep, 1.0, 0.0)
        col = jnp.sum(gs * onehot, axis=1, keepdims=True)
        rank = rank + jnp.where(col > gs, 1, 0)
        return rank + jnp.where((col == gs) & (ep < lane), 1, 0)

    rank = lax.fori_loop(0, _E, rank_step, jnp.zeros((_S, _E), jnp.int32))
    sel_ref[...] = jnp.where(rank < _KEXP, 1.0, 0.0)


def _route(h, gate2, gwt, gb2):
    return pl.pallas_call(
        _route_body,
        out_shape=(jax.ShapeDtypeStruct((_S, _D), jnp.float32),
                   jax.ShapeDtypeStruct((_S, _E), jnp.float32)),
        scratch_shapes=[pltpu.VMEM((_S, _D), jnp.float32)],
    )(h, gate2, gwt, gb2)


def _expert_body(hm_ref, sel_ref, ewt_ref, eb_ref, out_ref):
    sel = sel_ref[...]
    lane = lax.broadcasted_iota(jnp.int32, (_S, _E), 1)
    hmv = hm_ref[...]
    acc = jnp.zeros((_S, _CHE), jnp.float32)
    for e in range(_E):
        w = ewt_ref[e]
        ye = _dotk(hmv, w)
        ye = ye + eb_ref[e]
        sel_col = jnp.sum(jnp.where(lane == e, sel, 0.0), axis=1,
                          keepdims=True)
        acc = acc + sel_col * ye
    out_ref[...] = acc


def _experts(hm, sel, ewt, ebt):
    return pl.pallas_call(
        _expert_body,
        grid=(_NCHE,),
        in_specs=[
            pl.BlockSpec((_S, _D), lambda c: (0, 0)),
            pl.BlockSpec((_S, _E), lambda c: (0, 0)),
            pl.BlockSpec((_E, _D, _CHE), lambda c: (0, 0, c)),
            pl.BlockSpec((_E, 1, _CHE), lambda c: (0, 0, c)),
        ],
        out_specs=pl.BlockSpec((_S, _CHE), lambda c: (0, c)),
        out_shape=jax.ShapeDtypeStruct((_S, _D), jnp.float32),
    )(hm, sel, ewt, ebt)


def _head_body(h_ref, lnw_ref, lnb_ref, hwt_ref, hb_ref, out_ref, hn_sc):
    v = pl.program_id(0)

    @pl.when(v == 0)
    def _():
        h = h_ref[...]
        mu = jnp.mean(h, axis=1, keepdims=True)
        dlt = h - mu
        var = jnp.mean(dlt * dlt, axis=1, keepdims=True)
        hn = (h - mu) / jnp.sqrt(var + 1e-5)
        hn_sc[...] = hn * lnw_ref[...] + lnb_ref[...]

    logits = _dotk(hn_sc[...], hwt_ref[...])
    out_ref[...] = logits + hb_ref[...]


def _head_forward(h, ln_w2, ln_b2, head_wt, head_b_pad):
    vt = 256
    return pl.pallas_call(
        _head_body,
        grid=(_VPAD // vt,),
        in_specs=[
            pl.BlockSpec((_S, _D), lambda v: (0, 0)),
            pl.BlockSpec((1, _D), lambda v: (0, 0)),
            pl.BlockSpec((1, _D), lambda v: (0, 0)),
            pl.BlockSpec((_D, vt), lambda v: (0, v)),
            pl.BlockSpec((1, vt), lambda v: (0, v)),
        ],
        out_specs=pl.BlockSpec((_S, vt), lambda v: (0, v)),
        out_shape=jax.ShapeDtypeStruct((_S, _VPAD), jnp.float32),
        scratch_shapes=[pltpu.VMEM((_S, _D), jnp.float32)],
    )(h, ln_w2, ln_b2, head_wt, head_b_pad)


def kernel(x, emb, gate_w, expert_w, expert_b, gating_w, gating_b, ln_w,
           ln_b, head_w, head_b):
    idx = x.reshape(_S).astype(jnp.int32)
    h = _sc_gather(emb, idx)
    ewt = jnp.swapaxes(expert_w, -1, -2)
    for l in range(_L):
        hm, sel = _route(h, gate_w[l].reshape(1, _D).T,
                         gating_w[l].T, gating_b[l].reshape(1, _E))
        h = _experts(hm, sel, ewt[l], expert_b[l].reshape(_E, 1, _D))
    head_wt = jnp.pad(head_w, ((0, _VPAD - _V), (0, 0))).T
    head_b_pad = jnp.pad(head_b, (0, _VPAD - _V)).reshape(1, _VPAD)
    logits = _head_forward(h, ln_w.reshape(1, _D), ln_b.reshape(1, _D),
                           head_wt, head_b_pad)
    return logits[:, :_V].reshape(1, _S, _V)


# speedup vs baseline: 3.4805x; 1.1760x over previous
import functools

import jax
import jax.numpy as jnp
import numpy as np
from jax import lax
from jax.experimental import pallas as pl
from jax.experimental.pallas import tpu as pltpu
from jax.experimental.pallas import tpu_sc as plsc

_V = 1000
_D = 768
_L = 4
_E = 8
_S = 2048
_KTOK = max(1, int(_S * (1.0 / (1.0 + np.exp(-0.15)))))
_KEXP = max(1, int(0.8 * _E))
_VPAD = 1024
_CH = 256
_NCH = _D // _CH
_CHE = 256
_NCHE = _D // _CHE


def _dotk(a, b):
    acc = None
    for off in range(0, a.shape[1], 256):
        p = lax.dot_general(a[:, off:off + 256], b[:, off:off + 256],
                            (((1,), (1,)), ((), ())),
                            preferred_element_type=jnp.float32)
        acc = p if acc is None else acc + p
    return acc


def _sc_gather(emb, idx):
    info = plsc.get_sparse_core_info()
    nw = info.num_cores * info.num_subcores
    b_per_w = _S // nw
    mesh = plsc.VectorSubcoreMesh(core_axis_name="c", subcore_axis_name="s")

    @functools.partial(
        pl.kernel,
        mesh=mesh,
        out_type=jax.ShapeDtypeStruct((_S, _D), jnp.float32),
        scratch_types=[
            pltpu.VMEM((b_per_w,), jnp.int32),
            pltpu.VMEM((b_per_w, _D), jnp.float32),
            pltpu.SemaphoreType.DMA,
        ],
    )
    def k(emb_hbm, idx_hbm, out_hbm, idx_v, rows_v, sem):
        wid = lax.axis_index("s") * info.num_cores + lax.axis_index("c")
        base = wid * b_per_w
        pltpu.sync_copy(idx_hbm.at[pl.ds(base, b_per_w)], idx_v)
        pltpu.async_copy(emb_hbm.at[idx_v], rows_v, sem).wait()
        pltpu.sync_copy(rows_v, out_hbm.at[pl.ds(base, b_per_w)])

    return k(emb, idx)


def _f32_key(x):
    u = lax.bitcast_convert_type(x, jnp.uint32)
    neg = (u >> jnp.uint32(31)) > jnp.uint32(0)
    return jnp.where(neg, ~u, u | jnp.uint32(0x80000000))


def _kth_key(keys, k):
    def bit_step(i, m):
        cand = m | (jnp.uint32(1) << (jnp.uint32(31) - i.astype(jnp.uint32)))
        cnt = jnp.sum(jnp.where(keys >= cand, 1, 0).astype(jnp.int32))
        return jnp.where(cnt >= k, cand, m)

    return lax.fori_loop(0, 32, bit_step, jnp.uint32(0))


def _route_body(h_ref, gate_ref, gwt_ref, gb_ref, hm_ref, sel_ref, hn_sc):
    gate = gate_ref[...]
    s_col = jnp.zeros((_S, 1), jnp.float32)
    for c in range(_NCH):
        sl = slice(c * _CH, (c + 1) * _CH)
        h = h_ref[:, sl]
        mean = jnp.sum(h, axis=0, keepdims=True) * (1.0 / _S)
        dlt = h - mean
        var = jnp.sum(dlt * dlt, axis=0, keepdims=True) / (_S - 1.0)
        buf = (1.0 - 0.95) * mean
        hn = jnp.where(jnp.abs(dlt) > jnp.abs(buf), mean, h)
        hn = hn / jnp.sqrt(var + 1e-6)
        hn_sc[:, sl] = hn
        s_col = s_col + lax.dot_general(
            hn, gate[sl, :], (((1,), (0,)), ((), ())),
            preferred_element_type=jnp.float32)
    keys_col = _f32_key(s_col)
    thr = _kth_key(keys_col, _KTOK)
    tok = jnp.where(keys_col > thr, 1.0, 0.0)
    gs = gb_ref[...]
    gwt = gwt_ref[...]
    for c in range(_NCH):
        sl = slice(c * _CH, (c + 1) * _CH)
        hmv = hn_sc[:, sl] * tok
        hm_ref[:, sl] = hmv
        gs = gs + lax.dot_general(
            hmv, gwt[sl, :], (((1,), (0,)), ((), ())),
            preferred_element_type=jnp.float32)
    lane = lax.broadcasted_iota(jnp.int32, (_S, _E), 1)

    def rank_step(ep, rank):
        onehot = jnp.where(lane == ep, 1.0, 0.0)
        col = jnp.sum(gs * onehot, axis=1, keepdims=True)
        rank = rank + jnp.where(col > gs, 1, 0)
        return rank + jnp.where((col == gs) & (ep < lane), 1, 0)

    rank = lax.fori_loop(0, _E, rank_step, jnp.zeros((_S, _E), jnp.int32))
    sel_ref[...] = jnp.where(rank < _KEXP, 1.0, 0.0)


def _route(h, gate2, gwt, gb2):
    return pl.pallas_call(
        _route_body,
        out_shape=(jax.ShapeDtypeStruct((_S, _D), jnp.float32),
                   jax.ShapeDtypeStruct((_S, _E), jnp.float32)),
        scratch_shapes=[pltpu.VMEM((_S, _D), jnp.float32)],
    )(h, gate2, gwt, gb2)


def _expert_body(hm_ref, sel_ref, ew_ref, eb_ref, out_ref):
    sel = sel_ref[...]
    lane = lax.broadcasted_iota(jnp.int32, (_S, _E), 1)
    hmv = hm_ref[...]
    acc = jnp.zeros((_S, _CHE), jnp.float32)
    for e in range(_E):
        w = ew_ref[e]
        ye = _dotk(hmv, w)
        ye = ye + eb_ref[e]
        sel_col = jnp.sum(jnp.where(lane == e, sel, 0.0), axis=1,
                          keepdims=True)
        acc = acc + sel_col * ye
    out_ref[...] = acc


def _experts(hm, sel, ew, ebt):
    return pl.pallas_call(
        _expert_body,
        grid=(_NCHE,),
        in_specs=[
            pl.BlockSpec((_S, _D), lambda c: (0, 0)),
            pl.BlockSpec((_S, _E), lambda c: (0, 0)),
            pl.BlockSpec((_E, _CHE, _D), lambda c: (0, c, 0)),
            pl.BlockSpec((_E, 1, _CHE), lambda c: (0, 0, c)),
        ],
        out_specs=pl.BlockSpec((_S, _CHE), lambda c: (0, c)),
        out_shape=jax.ShapeDtypeStruct((_S, _D), jnp.float32),
    )(hm, sel, ew, ebt)


def _head_body(h_ref, lnw_ref, lnb_ref, hw_ref, hb_ref, out_ref, hn_sc):
    v = pl.program_id(0)

    @pl.when(v == 0)
    def _():
        h = h_ref[...]
        mu = jnp.mean(h, axis=1, keepdims=True)
        dlt = h - mu
        var = jnp.mean(dlt * dlt, axis=1, keepdims=True)
        hn = (h - mu) / jnp.sqrt(var + 1e-5)
        hn_sc[...] = hn * lnw_ref[...] + lnb_ref[...]

    logits = _dotk(hn_sc[...], hw_ref[...])
    out_ref[...] = logits + hb_ref[...]


def _head_forward(h, ln_w2, ln_b2, head_w_pad, head_b_pad):
    vt = 256
    return pl.pallas_call(
        _head_body,
        grid=(_VPAD // vt,),
        in_specs=[
            pl.BlockSpec((_S, _D), lambda v: (0, 0)),
            pl.BlockSpec((1, _D), lambda v: (0, 0)),
            pl.BlockSpec((1, _D), lambda v: (0, 0)),
            pl.BlockSpec((vt, _D), lambda v: (v, 0)),
            pl.BlockSpec((1, vt), lambda v: (0, v)),
        ],
        out_specs=pl.BlockSpec((_S, vt), lambda v: (0, v)),
        out_shape=jax.ShapeDtypeStruct((_S, _VPAD), jnp.float32),
        scratch_shapes=[pltpu.VMEM((_S, _D), jnp.float32)],
    )(h, ln_w2, ln_b2, head_w_pad, head_b_pad)


def kernel(x, emb, gate_w, expert_w, expert_b, gating_w, gating_b, ln_w,
           ln_b, head_w, head_b):
    idx = x.reshape(_S).astype(jnp.int32)
    h = _sc_gather(emb, idx)
    for l in range(_L):
        hm, sel = _route(h, gate_w[l].reshape(1, _D).T,
                         gating_w[l].T, gating_b[l].reshape(1, _E))
        h = _experts(hm, sel, expert_w[l], expert_b[l].reshape(_E, 1, _D))
    head_w_pad = jnp.pad(head_w, ((0, _VPAD - _V), (0, 0)))
    head_b_pad = jnp.pad(head_b, (0, _VPAD - _V)).reshape(1, _VPAD)
    logits = _head_forward(h, ln_w.reshape(1, _D), ln_b.reshape(1, _D),
                           head_w_pad, head_b_pad)
    return logits[:, :_V].reshape(1, _S, _V)
